# baseline (speedup 1.0000x reference)
import functools

import jax
import jax.numpy as jnp
from jax import lax
from jax.experimental import pallas as pl
from jax.experimental.pallas import tpu as pltpu
from jax.experimental.pallas import tpu_sc as plsc

_QROWS = 32
_NQ = 4
_LANES = 16


@functools.lru_cache(maxsize=None)
def _build_gather(NB, SEQ, V, D):
    info = plsc.get_sparse_core_info()
    nw = info.num_cores * info.num_subcores
    spw = NB // nw
    assert NB % nw == 0 and spw == _NQ * _QROWS
    nchunks = SEQ
    assert nchunks % 2 == 0 and nchunks >= 6
    coffs = list(range(0, D - _LANES + 1, _LANES))
    if coffs[-1] != D - _LANES:
        coffs.append(D - _LANES)

    mesh = plsc.VectorSubcoreMesh(core_axis_name="c", subcore_axis_name="s")

    @functools.partial(
        pl.kernel,
        mesh=mesh,
        compiler_params=pltpu.CompilerParams(needs_layout_passes=False),
        out_type=(jax.ShapeDtypeStruct((SEQ, D, NB), jnp.float32),
                  jax.ShapeDtypeStruct((1, D, NB), jnp.float32)),
        scratch_types=[
            pltpu.VMEM((1, spw), jnp.int32),
            pltpu.VMEM((1, spw), jnp.int32),
            pltpu.VMEM((_QROWS, D), jnp.float32),
            pltpu.VMEM((_QROWS, D), jnp.float32),
            pltpu.VMEM((_QROWS, D), jnp.float32),
            pltpu.VMEM((_QROWS, D), jnp.float32),
            pltpu.VMEM((1, D, spw), jnp.float32),
            pltpu.VMEM((1, D, spw), jnp.float32),
            pltpu.SemaphoreType.DMA,
            pltpu.SemaphoreType.DMA,
            pltpu.SemaphoreType.DMA,
            pltpu.SemaphoreType.DMA,
            pltpu.SemaphoreType.DMA,
            pltpu.SemaphoreType.DMA,
            pltpu.SemaphoreType.DMA,
            pltpu.SemaphoreType.DMA,
        ],
    )
    def body(x_hbm, table_hbm, out_hbm, dummy_hbm, ix0, ix1,
             q0, q1, q2, q3, t0, t1,
             is0, is1, gq0, gq1, gq2, gq3, ss0, ss1):
        wid = lax.axis_index("s") * info.num_cores + lax.axis_index("c")
        b0w = wid * spw

        ix = (ix0, ix1)
        isem = (is0, is1)
        qbuf = (q0, q1, q2, q3)
        gsem = (gq0, gq1, gq2, gq3)
        tbuf = (t0, t1)
        ssem = (ss0, ss1)
        iota = lax.iota(jnp.int32, _LANES)
        zero = jnp.zeros((_LANES,), jnp.int32)

        def fire_idx(c, w):
            pltpu.async_copy(x_hbm.at[pl.ds(c, 1), pl.ds(b0w, spw)],
                             ix[w], isem[w])

        def wait_idx(w):
            pltpu.make_async_copy(x_hbm.at[pl.ds(0, 1), pl.ds(0, spw)],
                                  ix[w], isem[w]).wait()

        def fire_gq(c_w, qi):
            for g in range(_QROWS // _LANES):
                b0 = qi * _QROWS + g * _LANES
                vv = plsc.load_gather(ix[c_w], [zero, iota + b0])
                for j in range(_LANES):
                    r = jnp.squeeze(lax.slice(vv, (j,), (j + 1,)))
                    pltpu.async_copy(
                        table_hbm.at[pl.ds(r, 1)],
                        qbuf[qi].at[pl.ds(g * _LANES + j, 1)],
                        gsem[qi])

        def wait_gq(qi):
            def w(k, carry):
                pltpu.make_async_copy(table_hbm.at[pl.ds(0, 1)],
                                      qbuf[qi].at[pl.ds(0, 1)],
                                      gsem[qi]).wait()
                return carry

            lax.fori_loop(0, _QROWS, w, 0)

        def transpose_q(qi, p):
            def row(b, carry):
                bvec = jnp.full((_LANES,), b, jnp.int32)
                col = bvec + qi * _QROWS
                for c0 in coffs:
                    vv = plsc.load_gather(qbuf[qi], [bvec, c0 + iota])
                    plsc.store_scatter(tbuf[p], [zero, c0 + iota, col], vv)
                return carry

            lax.fori_loop(0, _QROWS, row, 0)

        def fire_scat(c, p):
            pltpu.async_copy(
                tbuf[p],
                out_hbm.at[pl.ds(c, 1), :, pl.ds(b0w, spw)],
                ssem[p])

        def wait_scat(p):
            pltpu.make_async_copy(
                tbuf[p],
                out_hbm.at[pl.ds(0, 1), :, pl.ds(0, spw)],
                ssem[p]).wait()

        def do_chunk(c, p):
            cn = jnp.minimum(c + 1, nchunks - 1)
            fire_idx(cn, 1 - p)
            wait_scat(p)
            for qi in range(_NQ):
                wait_gq(qi)
                transpose_q(qi, p)
                if qi == 0:
                    wait_idx(1 - p)
                fire_gq(1 - p, qi)
            fire_scat(c, p)

        fire_idx(0, 0)
        wait_idx(0)
        for qi in range(_NQ):
            fire_gq(0, qi)
        for p in range(2):
            pltpu.async_copy(tbuf[p],
                             dummy_hbm.at[pl.ds(0, 1), :, pl.ds(b0w, spw)],
                             ssem[p])

        def iter_body(k, carry):
            do_chunk(2 * k, 0)
            do_chunk(2 * k + 1, 1)
            return carry

        lax.fori_loop(0, nchunks // 2, iter_body, 0)

        for qi in range(_NQ):
            wait_gq(qi)
        wait_scat(0)
        wait_scat(1)

    return body


def kernel(x, table):
    NB, SEQ = x.shape
    V, D = table.shape
    z, _ = _build_gather(NB, SEQ, V, D)(jnp.swapaxes(x, 0, 1), table)
    return jnp.transpose(z, (2, 0, 1))


# speedup vs baseline: 1.6085x; 1.6085x over previous
import functools

import jax
import jax.numpy as jnp
from jax import lax
from jax.experimental import pallas as pl
from jax.experimental.pallas import tpu as pltpu
from jax.experimental.pallas import tpu_sc as plsc

_QROWS = 32
_NQ = 4
_LANES = 16


@functools.lru_cache(maxsize=None)
def _build_gather(NB, SEQ, V, D):
    info = plsc.get_sparse_core_info()
    nw = info.num_cores * info.num_subcores
    spw = NB // nw
    assert NB % nw == 0 and spw == _NQ * _QROWS
    nchunks = SEQ
    assert nchunks % 2 == 0 and nchunks >= 6
    coffs = list(range(0, D - _LANES + 1, _LANES))
    if coffs[-1] != D - _LANES:
        coffs.append(D - _LANES)

    mesh = plsc.VectorSubcoreMesh(core_axis_name="c", subcore_axis_name="s")

    @functools.partial(
        pl.kernel,
        mesh=mesh,
        compiler_params=pltpu.CompilerParams(needs_layout_passes=False),
        out_type=(jax.ShapeDtypeStruct((SEQ, D, NB), jnp.float32),
                  jax.ShapeDtypeStruct((1, D, NB), jnp.float32)),
        scratch_types=[
            pltpu.VMEM((1, spw), jnp.int32),
            pltpu.VMEM((1, spw), jnp.int32),
            pltpu.VMEM((_QROWS, D), jnp.float32),
            pltpu.VMEM((_QROWS, D), jnp.float32),
            pltpu.VMEM((_QROWS, D), jnp.float32),
            pltpu.VMEM((_QROWS, D), jnp.float32),
            pltpu.VMEM((1, D, spw), jnp.float32),
            pltpu.VMEM((1, D, spw), jnp.float32),
            pltpu.SemaphoreType.DMA,
            pltpu.SemaphoreType.DMA,
            pltpu.SemaphoreType.DMA,
            pltpu.SemaphoreType.DMA,
            pltpu.SemaphoreType.DMA,
            pltpu.SemaphoreType.DMA,
            pltpu.SemaphoreType.DMA,
            pltpu.SemaphoreType.DMA,
        ],
    )
    def body(x_hbm, table_hbm, out_hbm, dummy_hbm, ix0, ix1,
             q0, q1, q2, q3, t0, t1,
             is0, is1, gq0, gq1, gq2, gq3, ss0, ss1):
        wid = lax.axis_index("s") * info.num_cores + lax.axis_index("c")
        b0w = wid * spw

        ix = (ix0, ix1)
        isem = (is0, is1)
        qbuf = (q0, q1, q2, q3)
        gsem = (gq0, gq1, gq2, gq3)
        tbuf = (t0, t1)
        ssem = (ss0, ss1)
        iota = lax.iota(jnp.int32, _LANES)
        zero = jnp.zeros((_LANES,), jnp.int32)

        def fire_idx(c, w):
            pltpu.async_copy(x_hbm.at[pl.ds(c, 1), pl.ds(b0w, spw)],
                             ix[w], isem[w])

        def wait_idx(w):
            pltpu.make_async_copy(x_hbm.at[pl.ds(0, 1), pl.ds(0, spw)],
                                  ix[w], isem[w]).wait()

        def fire_gq(c_w, qi):
            for g in range(_QROWS // _LANES):
                b0 = qi * _QROWS + g * _LANES
                vv = plsc.load_gather(ix[c_w], [zero, iota + b0])
                for j in range(_LANES):
                    r = jnp.squeeze(lax.slice(vv, (j,), (j + 1,)))
                    pltpu.async_copy(
                        table_hbm.at[pl.ds(r, 1)],
                        qbuf[qi].at[pl.ds(g * _LANES + j, 1)],
                        gsem[qi])

        def wait_gq(qi):
            def w(k, carry):
                pltpu.make_async_copy(table_hbm.at[pl.ds(0, 1)],
                                      qbuf[qi].at[pl.ds(0, 1)],
                                      gsem[qi]).wait()
                return carry

            lax.fori_loop(0, _QROWS, w, 0)

        nbg = _QROWS // _LANES
        ncg = len(coffs)
        cmax = D - _LANES

        def transpose_q(qi, p):
            def blk(k, carry):
                b0 = (k % nbg) * _LANES + qi * _QROWS
                c0 = jnp.minimum((k // nbg) * _LANES, cmax)
                cvec = c0 + iota
                for d in range(_LANES):
                    bvec = b0 + ((iota + d) & (_LANES - 1))
                    vv = plsc.load_gather(qbuf[qi],
                                          [bvec - qi * _QROWS, cvec])
                    plsc.store_scatter(tbuf[p], [zero, cvec, bvec], vv)
                return carry

            lax.fori_loop(0, nbg * ncg, blk, 0)

        def fire_scat(c, p):
            pltpu.async_copy(
                tbuf[p],
                out_hbm.at[pl.ds(c, 1), :, pl.ds(b0w, spw)],
                ssem[p])

        def wait_scat(p):
            pltpu.make_async_copy(
                tbuf[p],
                out_hbm.at[pl.ds(0, 1), :, pl.ds(0, spw)],
                ssem[p]).wait()

        def do_chunk(c, p):
            cn = jnp.minimum(c + 1, nchunks - 1)
            fire_idx(cn, 1 - p)
            wait_scat(p)
            for qi in range(_NQ):
                wait_gq(qi)
                transpose_q(qi, p)
                if qi == 0:
                    wait_idx(1 - p)
                fire_gq(1 - p, qi)
            fire_scat(c, p)

        fire_idx(0, 0)
        wait_idx(0)
        for qi in range(_NQ):
            fire_gq(0, qi)
        for p in range(2):
            pltpu.async_copy(tbuf[p],
                             dummy_hbm.at[pl.ds(0, 1), :, pl.ds(b0w, spw)],
                             ssem[p])

        def iter_body(k, carry):
            do_chunk(2 * k, 0)
            do_chunk(2 * k + 1, 1)
            return carry

        lax.fori_loop(0, nchunks // 2, iter_body, 0)

        for qi in range(_NQ):
            wait_gq(qi)
        wait_scat(0)
        wait_scat(1)

    return body


def kernel(x, table):
    NB, SEQ = x.shape
    V, D = table.shape
    z, _ = _build_gather(NB, SEQ, V, D)(jnp.swapaxes(x, 0, 1), table)
    return jnp.transpose(z, (2, 0, 1))


# speedup vs baseline: 1.6141x; 1.0035x over previous
import functools

import jax
import jax.numpy as jnp
from jax import lax
from jax.experimental import pallas as pl
from jax.experimental.pallas import tpu as pltpu
from jax.experimental.pallas import tpu_sc as plsc

_QROWS = 32
_NQ = 4
_LANES = 16


@functools.lru_cache(maxsize=None)
def _build_gather(NB, SEQ, V, D):
    info = plsc.get_sparse_core_info()
    nw = info.num_cores * info.num_subcores
    spw = NB // nw
    assert NB % nw == 0 and spw == _NQ * _QROWS
    nchunks = SEQ
    assert nchunks % 2 == 0 and nchunks >= 6
    coffs = list(range(0, D - _LANES + 1, _LANES))
    if coffs[-1] != D - _LANES:
        coffs.append(D - _LANES)

    mesh = plsc.VectorSubcoreMesh(core_axis_name="c", subcore_axis_name="s")

    @functools.partial(
        pl.kernel,
        mesh=mesh,
        compiler_params=pltpu.CompilerParams(needs_layout_passes=False),
        out_type=(jax.ShapeDtypeStruct((SEQ, D, NB), jnp.float32),
                  jax.ShapeDtypeStruct((1, D, NB), jnp.float32)),
        scratch_types=[
            pltpu.VMEM((1, spw), jnp.int32),
            pltpu.VMEM((1, spw), jnp.int32),
            pltpu.VMEM((_QROWS, D), jnp.float32),
            pltpu.VMEM((_QROWS, D), jnp.float32),
            pltpu.VMEM((_QROWS, D), jnp.float32),
            pltpu.VMEM((_QROWS, D), jnp.float32),
            pltpu.VMEM((1, D, spw), jnp.float32),
            pltpu.VMEM((1, D, spw), jnp.float32),
            pltpu.SemaphoreType.DMA,
            pltpu.SemaphoreType.DMA,
            pltpu.SemaphoreType.DMA,
            pltpu.SemaphoreType.DMA,
            pltpu.SemaphoreType.DMA,
            pltpu.SemaphoreType.DMA,
            pltpu.SemaphoreType.DMA,
            pltpu.SemaphoreType.DMA,
        ],
    )
    def body(x_hbm, table_hbm, out_hbm, dummy_hbm, ix0, ix1,
             q0, q1, q2, q3, t0, t1,
             is0, is1, gq0, gq1, gq2, gq3, ss0, ss1):
        wid = lax.axis_index("s") * info.num_cores + lax.axis_index("c")
        b0w = wid * spw

        ix = (ix0, ix1)
        isem = (is0, is1)
        qbuf = (q0, q1, q2, q3)
        gsem = (gq0, gq1, gq2, gq3)
        tbuf = (t0, t1)
        ssem = (ss0, ss1)
        iota = lax.iota(jnp.int32, _LANES)
        zero = jnp.zeros((_LANES,), jnp.int32)
        rots = [(iota + d) & (_LANES - 1) for d in range(_LANES)]

        def fire_idx(c, w):
            pltpu.async_copy(x_hbm.at[pl.ds(c, 1), pl.ds(b0w, spw)],
                             ix[w], isem[w])

        def wait_idx(w):
            pltpu.make_async_copy(x_hbm.at[pl.ds(0, 1), pl.ds(0, spw)],
                                  ix[w], isem[w]).wait()

        def fire_gq(c_w, qi):
            for g in range(_QROWS // _LANES):
                b0 = qi * _QROWS + g * _LANES
                vv = plsc.load_gather(ix[c_w], [zero, iota + b0])
                for j in range(_LANES):
                    r = jnp.squeeze(lax.slice(vv, (j,), (j + 1,)))
                    pltpu.async_copy(
                        table_hbm.at[pl.ds(r, 1)],
                        qbuf[qi].at[pl.ds(g * _LANES + j, 1)],
                        gsem[qi])

        def wait_gq(qi):
            def w(k, carry):
                pltpu.make_async_copy(table_hbm.at[pl.ds(0, 1)],
                                      qbuf[qi].at[pl.ds(0, 1)],
                                      gsem[qi]).wait()
                return carry

            lax.fori_loop(0, _QROWS, w, 0)

        nbg = _QROWS // _LANES
        ncg = len(coffs)
        cmax = D - _LANES

        def transpose_q(qi, p):
            def blk(k, carry):
                b0l = (k % nbg) * _LANES
                c0 = jnp.minimum((k // nbg) * _LANES, cmax)
                cvec = c0 + iota
                for d in range(_LANES):
                    bl = b0l + rots[d]
                    vv = plsc.load_gather(qbuf[qi], [bl, cvec])
                    plsc.store_scatter(tbuf[p],
                                       [zero, cvec, bl + qi * _QROWS], vv)
                return carry

            lax.fori_loop(0, nbg * ncg, blk, 0)

        def fire_scat(c, p):
            pltpu.async_copy(
                tbuf[p],
                out_hbm.at[pl.ds(c, 1), :, pl.ds(b0w, spw)],
                ssem[p])

        def wait_scat(p):
            pltpu.make_async_copy(
                tbuf[p],
                out_hbm.at[pl.ds(0, 1), :, pl.ds(0, spw)],
                ssem[p]).wait()

        def do_chunk(c, p):
            cn = jnp.minimum(c + 1, nchunks - 1)
            fire_idx(cn, 1 - p)
            wait_scat(p)
            for qi in range(_NQ):
                wait_gq(qi)
                transpose_q(qi, p)
                if qi == 0:
                    wait_idx(1 - p)
                fire_gq(1 - p, qi)
            fire_scat(c, p)

        fire_idx(0, 0)
        wait_idx(0)
        for qi in range(_NQ):
            fire_gq(0, qi)
        for p in range(2):
            pltpu.async_copy(tbuf[p],
                             dummy_hbm.at[pl.ds(0, 1), :, pl.ds(b0w, spw)],
                             ssem[p])

        def iter_body(k, carry):
            do_chunk(2 * k, 0)
            do_chunk(2 * k + 1, 1)
            return carry

        lax.fori_loop(0, nchunks // 2, iter_body, 0)

        for qi in range(_NQ):
            wait_gq(qi)
        wait_scat(0)
        wait_scat(1)

    return body


def kernel(x, table):
    NB, SEQ = x.shape
    V, D = table.shape
    z, _ = _build_gather(NB, SEQ, V, D)(jnp.swapaxes(x, 0, 1), table)
    return jnp.transpose(z, (2, 0, 1))


# speedup vs baseline: 1.7095x; 1.0591x over previous
import functools

import jax
import jax.numpy as jnp
from jax import lax
from jax.experimental import pallas as pl
from jax.experimental.pallas import tpu as pltpu
from jax.experimental.pallas import tpu_sc as plsc

_SPC = 2
_NBUF = 2
_LANES = 16


@functools.lru_cache(maxsize=None)
def _build_gather(NB, SEQ, V, D):
    info = plsc.get_sparse_core_info()
    nw = info.num_cores * info.num_subcores
    assert NB % (nw * _SPC * _NBUF) == 0
    s_per_w = NB // nw
    b_per_w = s_per_w * SEQ
    rows_c = _SPC * SEQ
    nchunks = s_per_w // _SPC
    niter = nchunks // _NBUF
    ngrp = rows_c // _LANES
    tail = rows_c - ngrp * _LANES

    mesh = plsc.VectorSubcoreMesh(core_axis_name="c", subcore_axis_name="s")

    @functools.partial(
        pl.kernel,
        mesh=mesh,
        out_type=jax.ShapeDtypeStruct((NB, SEQ, D), jnp.float32),
        scratch_types=[
            pltpu.VMEM((b_per_w + _LANES,), jnp.int32),
            pltpu.VMEM((_SPC, SEQ, D), jnp.float32),
            pltpu.VMEM((_SPC, SEQ, D), jnp.float32),
            pltpu.SemaphoreType.DMA,
            pltpu.SemaphoreType.DMA,
            pltpu.SemaphoreType.DMA,
            pltpu.SemaphoreType.DMA,
        ],
    )
    def body(x_hbm, table_hbm, out_hbm, idx_v, rows0, rows1, g0, g1, s0, s1):
        wid = lax.axis_index("s") * info.num_cores + lax.axis_index("c")
        base = wid * b_per_w
        pltpu.sync_copy(x_hbm.at[pl.ds(base, b_per_w)],
                        idx_v.at[pl.ds(0, b_per_w)])

        rows = (rows0, rows1)
        gsem = (g0, g1)
        ssem = (s0, s1)

        def enqueue(vv, j, dst_row, b):
            r = jnp.squeeze(lax.slice(vv, (j,), (j + 1,)))
            q = dst_row // SEQ
            s = dst_row - q * SEQ
            pltpu.async_copy(table_hbm.at[pl.ds(r, 1)],
                             rows[b].at[q].at[pl.ds(s, 1)], gsem[b])

        def fire_gather(c, b):
            def grp_body(g, carry):
                vv = idx_v[pl.ds(c * rows_c + g * _LANES, _LANES)]
                for j in range(_LANES):
                    enqueue(vv, j, g * _LANES + j, b)
                return carry

            lax.fori_loop(0, ngrp, grp_body, 0)
            if tail:
                vv = idx_v[pl.ds(c * rows_c + ngrp * _LANES, _LANES)]
                for j in range(tail):
                    enqueue(vv, j, ngrp * _LANES + j, b)

        def wait_gather(b):
            def row_body(k, carry):
                pltpu.make_async_copy(table_hbm.at[pl.ds(0, 1)],
                                      rows[b].at[0].at[pl.ds(0, 1)],
                                      gsem[b]).wait()
                return carry

            lax.fori_loop(0, rows_c, row_body, 0)

        def fire_scatter(c, b):
            s0_ = wid * s_per_w + c * _SPC
            for q in range(_SPC):
                pltpu.async_copy(rows[b].at[q], out_hbm.at[s0_ + q], ssem[b])

        def wait_scatter(b):
            for q in range(_SPC):
                pltpu.make_async_copy(rows[b].at[0], out_hbm.at[0],
                                      ssem[b]).wait()

        for b in range(_NBUF):
            fire_gather(b, b)

        def iter_body(i, carry):
            for b in range(_NBUF):
                c = _NBUF * i + b
                wait_gather(b)
                fire_scatter(c, b)
                wait_scatter(b)
                fire_gather(c + _NBUF, b)
            return carry

        lax.fori_loop(0, niter - 1, iter_body, 0)

        for b in range(_NBUF):
            wait_gather(b)
            fire_scatter(_NBUF * (niter - 1) + b, b)
        for b in range(_NBUF):
            wait_scatter(b)

    return body


def kernel(x, table):
    NB, SEQ = x.shape
    V, D = table.shape
    return _build_gather(NB, SEQ, V, D)(x.reshape(x.size), table)
